# Initial kernel scaffold; baseline (speedup 1.0000x reference)
import jax
import jax.numpy as jnp
from jax.experimental import pallas as pl


def kernel(tokens, wte, wpe):
    raise NotImplementedError("write your pallas kernel here")


# speedup vs baseline: 1.3312x; 1.3312x over previous
import functools

import jax
import jax.numpy as jnp
from jax import lax
from jax.experimental import pallas as pl
from jax.experimental.pallas import tpu as pltpu
from jax.experimental.pallas import tpu_sc as plsc

VOCAB = 100000
LENGTH = 2048
FEATURES = 768

_NC = 2
_NS = 16
_NW = _NC * _NS
_LANES = 16
_B_PER_W = LENGTH // _NW
_VECS_PER_ROW = FEATURES // _LANES


def _make_sc_kernel():
    mesh = plsc.VectorSubcoreMesh(core_axis_name="c", subcore_axis_name="s")

    @functools.partial(
        pl.kernel,
        mesh=mesh,
        out_type=jax.ShapeDtypeStruct((LENGTH, FEATURES), jnp.float32),
        scratch_types=[
            pltpu.VMEM((_B_PER_W,), jnp.int32),
            pltpu.VMEM((_B_PER_W, FEATURES), jnp.float32),
            pltpu.VMEM((_B_PER_W, FEATURES), jnp.float32),
            pltpu.SemaphoreType.DMA,
        ],
    )
    def emb_kernel(tokens_hbm, wte_hbm, wpe_hbm, out_hbm,
                   idx_v, rows_v, wpe_v, sem):
        wid = lax.axis_index("s") * _NC + lax.axis_index("c")
        base = wid * _B_PER_W
        pltpu.sync_copy(tokens_hbm.at[pl.ds(base, _B_PER_W)], idx_v)
        gather = pltpu.async_copy(wte_hbm.at[idx_v], rows_v, sem)
        pltpu.sync_copy(wpe_hbm.at[pl.ds(base, _B_PER_W)], wpe_v)
        gather.wait()

        def add_row(r, carry):
            for c in range(_VECS_PER_ROW):
                sl = pl.ds(c * _LANES, _LANES)
                rows_v[r, sl] = rows_v[r, sl] + wpe_v[r, sl]
            return carry

        lax.fori_loop(0, _B_PER_W, add_row, 0)
        pltpu.sync_copy(rows_v, out_hbm.at[pl.ds(base, _B_PER_W)])

    return emb_kernel


_emb_kernel = _make_sc_kernel()


def kernel(tokens, wte, wpe):
    return _emb_kernel(tokens.astype(jnp.int32), wte, wpe)


# speedup vs baseline: 1.3546x; 1.0176x over previous
import functools

import jax
import jax.numpy as jnp
from jax import lax
from jax.experimental import pallas as pl
from jax.experimental.pallas import tpu as pltpu
from jax.experimental.pallas import tpu_sc as plsc

VOCAB = 100000
LENGTH = 2048
FEATURES = 768

_NC = 2
_NS = 16
_NW = _NC * _NS
_LANES = 16
_B_PER_W = LENGTH // _NW
_CHUNK = _LANES
_NCHUNK = _B_PER_W // _CHUNK
_VECS_PER_ROW = FEATURES // _LANES


def _make_sc_kernel():
    mesh = plsc.VectorSubcoreMesh(core_axis_name="c", subcore_axis_name="s")

    @functools.partial(
        pl.kernel,
        mesh=mesh,
        out_type=jax.ShapeDtypeStruct((LENGTH, FEATURES), jnp.float32),
        scratch_types=[
            pltpu.VMEM((_B_PER_W,), jnp.int32),
            pltpu.VMEM((_B_PER_W, FEATURES), jnp.float32),
            pltpu.VMEM((_B_PER_W, FEATURES), jnp.float32),
        ]
        + [pltpu.SemaphoreType.DMA] * (2 * _NCHUNK + 1),
    )
    def emb_kernel(tokens_hbm, wte_hbm, wpe_hbm, out_hbm,
                   idx_v, rows_v, wpe_v, *sems):
        gsems = sems[:_NCHUNK]
        wsems = sems[_NCHUNK:2 * _NCHUNK]
        ssem = sems[2 * _NCHUNK]
        wid = lax.axis_index("s") * _NC + lax.axis_index("c")
        base = wid * _B_PER_W

        pltpu.sync_copy(tokens_hbm.at[pl.ds(base, _B_PER_W)], idx_v)

        gathers, wcopies = [], []
        for k in range(_NCHUNK):
            row0 = k * _CHUNK
            idxs = idx_v[pl.ds(row0, _CHUNK)]
            gathers.append(pltpu.async_copy(
                wte_hbm.at[idxs], rows_v.at[pl.ds(row0, _CHUNK)], gsems[k]))
            wcopies.append(pltpu.async_copy(
                wpe_hbm.at[pl.ds(base + row0, _CHUNK)],
                wpe_v.at[pl.ds(row0, _CHUNK)], wsems[k]))

        stores = []
        for k in range(_NCHUNK):
            row0 = k * _CHUNK
            gathers[k].wait()
            wcopies[k].wait()

            def add_row(r, carry):
                for c in range(_VECS_PER_ROW):
                    sl = pl.ds(c * _LANES, _LANES)
                    plsc.addupdate(rows_v.at[r, sl], wpe_v[r, sl])
                return carry

            lax.fori_loop(row0, row0 + _CHUNK, add_row, 0)
            stores.append(pltpu.async_copy(
                rows_v.at[pl.ds(row0, _CHUNK)],
                out_hbm.at[pl.ds(base + row0, _CHUNK)], ssem))
        for st in stores:
            st.wait()

    return emb_kernel


_emb_kernel = _make_sc_kernel()


def kernel(tokens, wte, wpe):
    return _emb_kernel(tokens.astype(jnp.int32), wte, wpe)
